# baseline (speedup 1.0000x reference)
import jax
import jax.numpy as jnp
from jax.experimental import pallas as pl
from jax.experimental.pallas import tpu as pltpu

LANE = 128


def _linear_kernel(x_ref, w_ref, b_ref, o_ref):
    d_out = o_ref.shape[-1]
    acc = jnp.dot(x_ref[...], w_ref[...], preferred_element_type=jnp.float32)
    o_ref[...] = (acc + b_ref[...])[:, :d_out].astype(o_ref.dtype)


def kernel(x, w, b):
    B, D_in = x.shape
    D_out = w.shape[0]

    w_t = jnp.zeros((D_in, LANE), x.dtype).at[:, :D_out].set(w.T.astype(x.dtype))
    b_p = jnp.zeros((1, LANE), x.dtype).at[0, :D_out].set(b.astype(x.dtype))

    tb = 2048
    n_tiles = B // tb

    out = pl.pallas_call(
        _linear_kernel,
        out_shape=jax.ShapeDtypeStruct((B, D_out), x.dtype),
        grid_spec=pltpu.PrefetchScalarGridSpec(
            num_scalar_prefetch=0,
            grid=(n_tiles,),
            in_specs=[
                pl.BlockSpec((tb, D_in), lambda i: (i, 0)),
                pl.BlockSpec((D_in, LANE), lambda i: (0, 0)),
                pl.BlockSpec((1, LANE), lambda i: (0, 0)),
            ],
            out_specs=pl.BlockSpec((tb, D_out), lambda i: (i, 0)),
        ),
        compiler_params=pltpu.CompilerParams(
            dimension_semantics=("parallel",),
        ),
    )(x, w_t, b_p)
    return out


# speedup vs baseline: 1.2968x; 1.2968x over previous
import jax
import jax.numpy as jnp
from jax.experimental import pallas as pl
from jax.experimental.pallas import tpu as pltpu

LANE = 128


def _linear_kernel(x_ref, w_ref, b_ref, o_ref):
    d_out = o_ref.shape[-1]
    acc = jnp.dot(x_ref[...], w_ref[...], preferred_element_type=jnp.float32)
    o_ref[...] = (acc + b_ref[...])[:, :d_out].astype(o_ref.dtype)


def kernel(x, w, b):
    B, D_in = x.shape
    D_out = w.shape[0]

    w_t = jnp.zeros((D_in, LANE), x.dtype).at[:, :D_out].set(w.T.astype(x.dtype))
    b_p = jnp.zeros((1, LANE), x.dtype).at[0, :D_out].set(b.astype(x.dtype))

    tb = 8192
    n_tiles = B // tb

    out = pl.pallas_call(
        _linear_kernel,
        out_shape=jax.ShapeDtypeStruct((B, D_out), x.dtype),
        grid_spec=pltpu.PrefetchScalarGridSpec(
            num_scalar_prefetch=0,
            grid=(n_tiles,),
            in_specs=[
                pl.BlockSpec((tb, D_in), lambda i: (i, 0)),
                pl.BlockSpec((D_in, LANE), lambda i: (0, 0)),
                pl.BlockSpec((1, LANE), lambda i: (0, 0)),
            ],
            out_specs=pl.BlockSpec((tb, D_out), lambda i: (i, 0)),
        ),
        compiler_params=pltpu.CompilerParams(
            dimension_semantics=("parallel",),
        ),
    )(x, w_t, b_p)
    return out


# speedup vs baseline: 1.3008x; 1.0031x over previous
import jax
import jax.numpy as jnp
from jax.experimental import pallas as pl
from jax.experimental.pallas import tpu as pltpu

LANE = 128


def _linear_kernel(x_ref, w_ref, b_ref, o_ref):
    d_out = o_ref.shape[-1]
    acc = jnp.dot(x_ref[...].astype(jnp.bfloat16), w_ref[...],
                  preferred_element_type=jnp.float32)
    o_ref[...] = (acc + b_ref[...])[:, :d_out].astype(o_ref.dtype)


def kernel(x, w, b):
    B, D_in = x.shape
    D_out = w.shape[0]

    w_t = jnp.zeros((D_in, LANE), jnp.bfloat16).at[:, :D_out].set(
        w.T.astype(jnp.bfloat16))
    b_p = jnp.zeros((1, LANE), x.dtype).at[0, :D_out].set(b.astype(x.dtype))

    tb = 8192
    n_tiles = B // tb

    out = pl.pallas_call(
        _linear_kernel,
        out_shape=jax.ShapeDtypeStruct((B, D_out), x.dtype),
        grid_spec=pltpu.PrefetchScalarGridSpec(
            num_scalar_prefetch=0,
            grid=(n_tiles,),
            in_specs=[
                pl.BlockSpec((tb, D_in), lambda i: (i, 0)),
                pl.BlockSpec((D_in, LANE), lambda i: (0, 0)),
                pl.BlockSpec((1, LANE), lambda i: (0, 0)),
            ],
            out_specs=pl.BlockSpec((tb, D_out), lambda i: (i, 0)),
        ),
        compiler_params=pltpu.CompilerParams(
            dimension_semantics=("parallel",),
        ),
    )(x, w_t, b_p)
    return out


# speedup vs baseline: 1.3452x; 1.0341x over previous
import jax
import jax.numpy as jnp
from jax.experimental import pallas as pl
from jax.experimental.pallas import tpu as pltpu

LANE = 128


def _linear_kernel(x_ref, w_ref, b_ref, o_ref):
    d_out = o_ref.shape[-1]
    acc = jnp.dot(x_ref[...], w_ref[...], preferred_element_type=jnp.float32)
    o_ref[...] = (acc + b_ref[...])[:, :d_out].astype(o_ref.dtype)


def kernel(x, w, b):
    B, D_in = x.shape
    D_out = w.shape[0]

    w_t = jnp.zeros((D_in, LANE), x.dtype).at[:, :D_out].set(w.T.astype(x.dtype))
    b_p = jnp.zeros((1, LANE), x.dtype).at[0, :D_out].set(b.astype(x.dtype))

    tb = 16384
    n_tiles = B // tb

    out = pl.pallas_call(
        _linear_kernel,
        out_shape=jax.ShapeDtypeStruct((B, D_out), x.dtype),
        grid_spec=pltpu.PrefetchScalarGridSpec(
            num_scalar_prefetch=0,
            grid=(n_tiles,),
            in_specs=[
                pl.BlockSpec((tb, D_in), lambda i: (i, 0)),
                pl.BlockSpec((D_in, LANE), lambda i: (0, 0)),
                pl.BlockSpec((1, LANE), lambda i: (0, 0)),
            ],
            out_specs=pl.BlockSpec((tb, D_out), lambda i: (i, 0)),
        ),
        compiler_params=pltpu.CompilerParams(
            dimension_semantics=("parallel",),
        ),
    )(x, w_t, b_p)
    return out
